# Initial kernel scaffold; baseline (speedup 1.0000x reference)
import jax
import jax.numpy as jnp
from jax.experimental import pallas as pl


def kernel(state, neuron_id, neuron_key, state_w1, state_b1, state_gs1, state_gb1, state_w2, state_b2, state_gs2, state_gb2, msg_w1, msg_b1, msg_gs1, msg_gb1, msg_w2, msg_b2, msg_gs2, msg_gb2, conn_idx, cell_to_group):
    raise NotImplementedError("write your pallas kernel here")


# speedup vs baseline: 33.9458x; 33.9458x over previous
import functools

import jax
import jax.numpy as jnp
from jax import lax
from jax.experimental import pallas as pl

D_N = 64
N_CELLS = 1024
C_N = 64
K = 16
HS = 128
HM = 128
HEADS = 4
DH = D_N // HEADS

B = 8


def _fwd_kernel(state_ref, nid_ref, nk_ref, conn_ref,
                mw1a_ref, mw1s_ref, mb1_ref, mgs1_ref, mgb1_ref,
                mw2_ref, mb2_ref, mgs2_ref, mgb2_ref,
                sw1s_ref, sw1m_ref, sb1_ref, sgs1_ref, sgb1_ref,
                sw2_ref, sb2_ref, sgs2_ref, sgb2_ref,
                out_ref):
    st = state_ref[...]
    x = st + nid_ref[...]
    q = nk_ref[...]
    idx = conn_ref[...]

    j = lax.broadcasted_iota(jnp.int32, (B, C_N, C_N), 2)
    m = j < 0
    for k in range(K):
        m = m | (j == idx[:, :, k][:, :, None])
    bias = jnp.where(m, 0.0, -1e30).astype(jnp.float32)

    h_iota = lax.broadcasted_iota(jnp.int32, (HEADS, D_N), 0)
    d_head = lax.broadcasted_iota(jnp.int32, (HEADS, D_N), 1) // DH
    hm = h_iota == d_head

    scale = 1.0 / (DH ** 0.5)
    agg = jnp.zeros_like(st)
    for h in range(HEADS):
        wh = jnp.where(hm[h][None, None, :], x, 0.0)
        s = lax.dot_general(q, wh, (((2,), (2,)), ((0,), (0,))),
                            preferred_element_type=jnp.float32)
        s = s * scale + bias
        s = s - jnp.max(s, axis=2, keepdims=True)
        e = jnp.exp(s)
        att = e / jnp.sum(e, axis=2, keepdims=True)
        agg = agg + lax.dot_general(att, wh, (((2,), (1,)), ((0,), (0,))),
                                    preferred_element_type=jnp.float32)

    h1 = jnp.einsum('bcd,hd->bch', agg, mw1a_ref[...],
                    preferred_element_type=jnp.float32)
    h1 = h1 + jnp.einsum('bcd,hd->bch', st, mw1s_ref[...],
                         preferred_element_type=jnp.float32)
    h1 = jnp.maximum(h1 + mb1_ref[...], 0.0)
    h1 = h1 * mgs1_ref[...][:, None, :] + mgb1_ref[...][:, None, :]
    msg = jnp.einsum('bch,dh->bcd', h1, mw2_ref[...],
                     preferred_element_type=jnp.float32) + mb2_ref[...]
    msg = msg * mgs2_ref[...][:, None, :] + mgb2_ref[...][:, None, :]

    h2 = jnp.einsum('bcd,hd->bch', st, sw1s_ref[...],
                    preferred_element_type=jnp.float32)
    h2 = h2 + jnp.einsum('bcd,hd->bch', msg, sw1m_ref[...],
                         preferred_element_type=jnp.float32)
    h2 = jnp.maximum(h2 + sb1_ref[...], 0.0)
    h2 = h2 * sgs1_ref[...][:, None, :] + sgb1_ref[...][:, None, :]
    delta = jnp.einsum('bch,dh->bcd', h2, sw2_ref[...],
                       preferred_element_type=jnp.float32) + sb2_ref[...]
    delta = delta * sgs2_ref[...][:, None, :] + sgb2_ref[...][:, None, :]

    out_ref[...] = st + delta


@jax.jit
def kernel(state, neuron_id, neuron_key,
           state_w1, state_b1, state_gs1, state_gb1, state_w2, state_b2,
           state_gs2, state_gb2,
           msg_w1, msg_b1, msg_gs1, msg_gb1, msg_w2, msg_b2, msg_gs2, msg_gb2,
           conn_idx, cell_to_group):
    g = cell_to_group
    mw1a = msg_w1[:, :D_N]
    mw1s = msg_w1[:, D_N:]
    sw1s = state_w1[:, :D_N]
    sw1m = state_w1[:, D_N:]
    mb1 = msg_b1.reshape(1, HM)
    mb2 = msg_b2.reshape(1, D_N)
    sb1 = state_b1.reshape(1, HS)
    sb2 = state_b2.reshape(1, D_N)
    mgs1, mgb1 = msg_gs1[g], msg_gb1[g]
    mgs2, mgb2 = msg_gs2[g], msg_gb2[g]
    sgs1, sgb1 = state_gs1[g], state_gb1[g]
    sgs2, sgb2 = state_gs2[g], state_gb2[g]

    grid = (N_CELLS // B,)
    cell_blk = pl.BlockSpec((B, C_N, D_N), lambda i: (i, 0, 0))
    conn_blk = pl.BlockSpec((B, C_N, K), lambda i: (i, 0, 0))

    def full2(a):
        return pl.BlockSpec(a.shape, lambda i: (0, 0))

    def row_blk(a):
        return pl.BlockSpec((B, a.shape[1]), lambda i: (i, 0))

    in_specs = [cell_blk, cell_blk, cell_blk, conn_blk,
                full2(mw1a), full2(mw1s), full2(mb1), row_blk(mgs1), row_blk(mgb1),
                full2(msg_w2), full2(mb2), row_blk(mgs2), row_blk(mgb2),
                full2(sw1s), full2(sw1m), full2(sb1), row_blk(sgs1), row_blk(sgb1),
                full2(state_w2), full2(sb2), row_blk(sgs2), row_blk(sgb2)]

    return pl.pallas_call(
        _fwd_kernel,
        grid=grid,
        in_specs=in_specs,
        out_specs=cell_blk,
        out_shape=jax.ShapeDtypeStruct((N_CELLS, C_N, D_N), jnp.float32),
    )(state, neuron_id, neuron_key, conn_idx,
      mw1a, mw1s, mb1, mgs1, mgb1, msg_w2, mb2, mgs2, mgb2,
      sw1s, sw1m, sb1, sgs1, sgb1, state_w2, sb2, sgs2, sgb2)


# speedup vs baseline: 37.3874x; 1.1014x over previous
import functools

import jax
import jax.numpy as jnp
from jax import lax
from jax.experimental import pallas as pl

D_N = 64
N_CELLS = 1024
C_N = 64
K = 16
HS = 128
HM = 128
HEADS = 4
DH = D_N // HEADS

B = 8


def _fwd_kernel(state_ref, nid_ref, nk_ref, conn_ref,
                mw1a_ref, mw1s_ref, mb1_ref, mgs1_ref, mgb1_ref,
                mw2_ref, mb2_ref, mgs2_ref, mgb2_ref,
                sw1s_ref, sw1m_ref, sb1_ref, sgs1_ref, sgb1_ref,
                sw2_ref, sb2_ref, sgs2_ref, sgb2_ref,
                out_ref):
    st = state_ref[...]
    x = st + nid_ref[...]
    q = nk_ref[...]
    idx = conn_ref[...]

    j = lax.broadcasted_iota(jnp.int32, (B, C_N, C_N), 2)
    m = j < 0
    for k in range(K):
        m = m | (j == idx[:, :, k][:, :, None])
    bias = jnp.where(m, 0.0, -1e30).astype(jnp.float32)

    h_iota = lax.broadcasted_iota(jnp.int32, (HEADS, D_N), 0)
    d_head = lax.broadcasted_iota(jnp.int32, (HEADS, D_N), 1) // DH
    hm = h_iota == d_head

    q = q * (1.0 / (DH ** 0.5))
    agg = jnp.zeros_like(st)
    for h in range(HEADS):
        wh = jnp.where(hm[h][None, None, :], x, 0.0)
        s = lax.dot_general(q, wh, (((2,), (2,)), ((0,), (0,))),
                            preferred_element_type=jnp.float32)
        e = jnp.exp(jnp.minimum(s + bias, 80.0))
        r = 1.0 / jnp.sum(e, axis=2, keepdims=True)
        agg = agg + lax.dot_general(e, wh, (((2,), (1,)), ((0,), (0,))),
                                    preferred_element_type=jnp.float32) * r

    h1 = jnp.einsum('bcd,hd->bch', agg, mw1a_ref[...],
                    preferred_element_type=jnp.float32)
    h1 = h1 + jnp.einsum('bcd,hd->bch', st, mw1s_ref[...],
                         preferred_element_type=jnp.float32)
    h1 = jnp.maximum(h1 + mb1_ref[...], 0.0)
    h1 = h1 * mgs1_ref[...][:, None, :] + mgb1_ref[...][:, None, :]
    msg = jnp.einsum('bch,dh->bcd', h1, mw2_ref[...],
                     preferred_element_type=jnp.float32) + mb2_ref[...]
    msg = msg * mgs2_ref[...][:, None, :] + mgb2_ref[...][:, None, :]

    h2 = jnp.einsum('bcd,hd->bch', st, sw1s_ref[...],
                    preferred_element_type=jnp.float32)
    h2 = h2 + jnp.einsum('bcd,hd->bch', msg, sw1m_ref[...],
                         preferred_element_type=jnp.float32)
    h2 = jnp.maximum(h2 + sb1_ref[...], 0.0)
    h2 = h2 * sgs1_ref[...][:, None, :] + sgb1_ref[...][:, None, :]
    delta = jnp.einsum('bch,dh->bcd', h2, sw2_ref[...],
                       preferred_element_type=jnp.float32) + sb2_ref[...]
    delta = delta * sgs2_ref[...][:, None, :] + sgb2_ref[...][:, None, :]

    out_ref[...] = st + delta


@jax.jit
def kernel(state, neuron_id, neuron_key,
           state_w1, state_b1, state_gs1, state_gb1, state_w2, state_b2,
           state_gs2, state_gb2,
           msg_w1, msg_b1, msg_gs1, msg_gb1, msg_w2, msg_b2, msg_gs2, msg_gb2,
           conn_idx, cell_to_group):
    g = cell_to_group
    mw1a = msg_w1[:, :D_N]
    mw1s = msg_w1[:, D_N:]
    sw1s = state_w1[:, :D_N]
    sw1m = state_w1[:, D_N:]
    mb1 = msg_b1.reshape(1, HM)
    mb2 = msg_b2.reshape(1, D_N)
    sb1 = state_b1.reshape(1, HS)
    sb2 = state_b2.reshape(1, D_N)
    mgs1, mgb1 = msg_gs1[g], msg_gb1[g]
    mgs2, mgb2 = msg_gs2[g], msg_gb2[g]
    sgs1, sgb1 = state_gs1[g], state_gb1[g]
    sgs2, sgb2 = state_gs2[g], state_gb2[g]

    grid = (N_CELLS // B,)
    cell_blk = pl.BlockSpec((B, C_N, D_N), lambda i: (i, 0, 0))
    conn_blk = pl.BlockSpec((B, C_N, K), lambda i: (i, 0, 0))

    def full2(a):
        return pl.BlockSpec(a.shape, lambda i: (0, 0))

    def row_blk(a):
        return pl.BlockSpec((B, a.shape[1]), lambda i: (i, 0))

    in_specs = [cell_blk, cell_blk, cell_blk, conn_blk,
                full2(mw1a), full2(mw1s), full2(mb1), row_blk(mgs1), row_blk(mgb1),
                full2(msg_w2), full2(mb2), row_blk(mgs2), row_blk(mgb2),
                full2(sw1s), full2(sw1m), full2(sb1), row_blk(sgs1), row_blk(sgb1),
                full2(state_w2), full2(sb2), row_blk(sgs2), row_blk(sgb2)]

    return pl.pallas_call(
        _fwd_kernel,
        grid=grid,
        in_specs=in_specs,
        out_specs=cell_blk,
        out_shape=jax.ShapeDtypeStruct((N_CELLS, C_N, D_N), jnp.float32),
    )(state, neuron_id, neuron_key, conn_idx,
      mw1a, mw1s, mb1, mgs1, mgb1, msg_w2, mb2, mgs2, mgb2,
      sw1s, sw1m, sb1, sgs1, sgb1, state_w2, sb2, sgs2, sgb2)


# speedup vs baseline: 41.1383x; 1.1003x over previous
import functools

import jax
import jax.numpy as jnp
from jax import lax
from jax.experimental import pallas as pl

D_N = 64
N_CELLS = 1024
C_N = 64
K = 16
HS = 128
HM = 128
HEADS = 4
DH = D_N // HEADS

B = 16


def _fwd_kernel(state_ref, nid_ref, nk_ref, conn_ref,
                mw1a_ref, mw1s_ref, mb1_ref, mgs1_ref, mgb1_ref,
                mw2_ref, mb2_ref, mgs2_ref, mgb2_ref,
                sw1s_ref, sw1m_ref, sb1_ref, sgs1_ref, sgb1_ref,
                sw2_ref, sb2_ref, sgs2_ref, sgb2_ref,
                out_ref):
    st = state_ref[...]
    x = st + nid_ref[...]
    q = nk_ref[...]
    idx = conn_ref[...]

    j = lax.broadcasted_iota(jnp.int32, (B, C_N, C_N), 2)
    m = j < 0
    for k in range(K):
        m = m | (j == idx[:, :, k][:, :, None])
    bias = jnp.where(m, 0.0, -1e30).astype(jnp.float32)

    h_iota = lax.broadcasted_iota(jnp.int32, (HEADS, D_N), 0)
    d_head = lax.broadcasted_iota(jnp.int32, (HEADS, D_N), 1) // DH
    hm = h_iota == d_head

    q = q * (1.0 / (DH ** 0.5))
    agg = jnp.zeros_like(st)
    for h in range(HEADS):
        wh = jnp.where(hm[h][None, None, :], x, 0.0)
        s = lax.dot_general(q, wh, (((2,), (2,)), ((0,), (0,))),
                            preferred_element_type=jnp.float32)
        e = jnp.exp(jnp.minimum(s + bias, 80.0))
        r = 1.0 / jnp.sum(e, axis=2, keepdims=True)
        agg = agg + lax.dot_general(e, wh, (((2,), (1,)), ((0,), (0,))),
                                    preferred_element_type=jnp.float32) * r

    h1 = jnp.einsum('bcd,hd->bch', agg, mw1a_ref[...],
                    preferred_element_type=jnp.float32)
    h1 = h1 + jnp.einsum('bcd,hd->bch', st, mw1s_ref[...],
                         preferred_element_type=jnp.float32)
    h1 = jnp.maximum(h1 + mb1_ref[...], 0.0)
    h1 = h1 * mgs1_ref[...][:, None, :] + mgb1_ref[...][:, None, :]
    msg = jnp.einsum('bch,dh->bcd', h1, mw2_ref[...],
                     preferred_element_type=jnp.float32) + mb2_ref[...]
    msg = msg * mgs2_ref[...][:, None, :] + mgb2_ref[...][:, None, :]

    h2 = jnp.einsum('bcd,hd->bch', st, sw1s_ref[...],
                    preferred_element_type=jnp.float32)
    h2 = h2 + jnp.einsum('bcd,hd->bch', msg, sw1m_ref[...],
                         preferred_element_type=jnp.float32)
    h2 = jnp.maximum(h2 + sb1_ref[...], 0.0)
    h2 = h2 * sgs1_ref[...][:, None, :] + sgb1_ref[...][:, None, :]
    delta = jnp.einsum('bch,dh->bcd', h2, sw2_ref[...],
                       preferred_element_type=jnp.float32) + sb2_ref[...]
    delta = delta * sgs2_ref[...][:, None, :] + sgb2_ref[...][:, None, :]

    out_ref[...] = st + delta


@jax.jit
def kernel(state, neuron_id, neuron_key,
           state_w1, state_b1, state_gs1, state_gb1, state_w2, state_b2,
           state_gs2, state_gb2,
           msg_w1, msg_b1, msg_gs1, msg_gb1, msg_w2, msg_b2, msg_gs2, msg_gb2,
           conn_idx, cell_to_group):
    g = cell_to_group
    mw1a = msg_w1[:, :D_N]
    mw1s = msg_w1[:, D_N:]
    sw1s = state_w1[:, :D_N]
    sw1m = state_w1[:, D_N:]
    mb1 = msg_b1.reshape(1, HM)
    mb2 = msg_b2.reshape(1, D_N)
    sb1 = state_b1.reshape(1, HS)
    sb2 = state_b2.reshape(1, D_N)
    mgs1, mgb1 = msg_gs1[g], msg_gb1[g]
    mgs2, mgb2 = msg_gs2[g], msg_gb2[g]
    sgs1, sgb1 = state_gs1[g], state_gb1[g]
    sgs2, sgb2 = state_gs2[g], state_gb2[g]

    grid = (N_CELLS // B,)
    cell_blk = pl.BlockSpec((B, C_N, D_N), lambda i: (i, 0, 0))
    conn_blk = pl.BlockSpec((B, C_N, K), lambda i: (i, 0, 0))

    def full2(a):
        return pl.BlockSpec(a.shape, lambda i: (0, 0))

    def row_blk(a):
        return pl.BlockSpec((B, a.shape[1]), lambda i: (i, 0))

    in_specs = [cell_blk, cell_blk, cell_blk, conn_blk,
                full2(mw1a), full2(mw1s), full2(mb1), row_blk(mgs1), row_blk(mgb1),
                full2(msg_w2), full2(mb2), row_blk(mgs2), row_blk(mgb2),
                full2(sw1s), full2(sw1m), full2(sb1), row_blk(sgs1), row_blk(sgb1),
                full2(state_w2), full2(sb2), row_blk(sgs2), row_blk(sgb2)]

    return pl.pallas_call(
        _fwd_kernel,
        grid=grid,
        in_specs=in_specs,
        out_specs=cell_blk,
        out_shape=jax.ShapeDtypeStruct((N_CELLS, C_N, D_N), jnp.float32),
    )(state, neuron_id, neuron_key, conn_idx,
      mw1a, mw1s, mb1, mgs1, mgb1, msg_w2, mb2, mgs2, mgb2,
      sw1s, sw1m, sb1, sgs1, sgb1, state_w2, sb2, sgs2, sgb2)


# speedup vs baseline: 42.1567x; 1.0248x over previous
import functools

import jax
import jax.numpy as jnp
from jax import lax
from jax.experimental import pallas as pl

D_N = 64
N_CELLS = 1024
C_N = 64
K = 16
HS = 128
HM = 128
HEADS = 4
DH = D_N // HEADS

B = 32


def _fwd_kernel(state_ref, nid_ref, nk_ref, conn_ref,
                mw1a_ref, mw1s_ref, mb1_ref, mgs1_ref, mgb1_ref,
                mw2_ref, mb2_ref, mgs2_ref, mgb2_ref,
                sw1s_ref, sw1m_ref, sb1_ref, sgs1_ref, sgb1_ref,
                sw2_ref, sb2_ref, sgs2_ref, sgb2_ref,
                out_ref):
    st = state_ref[...]
    x = st + nid_ref[...]
    q = nk_ref[...]
    idx = conn_ref[...]

    j = lax.broadcasted_iota(jnp.int32, (B, C_N, C_N), 2)
    m = j < 0
    for k in range(K):
        m = m | (j == idx[:, :, k][:, :, None])
    bias = jnp.where(m, 0.0, -1e30).astype(jnp.float32)

    h_iota = lax.broadcasted_iota(jnp.int32, (HEADS, D_N), 0)
    d_head = lax.broadcasted_iota(jnp.int32, (HEADS, D_N), 1) // DH
    hm = h_iota == d_head

    q = q * (1.0 / (DH ** 0.5))
    agg = jnp.zeros_like(st)
    for h in range(HEADS):
        wh = jnp.where(hm[h][None, None, :], x, 0.0)
        s = lax.dot_general(q, wh, (((2,), (2,)), ((0,), (0,))),
                            preferred_element_type=jnp.float32)
        e = jnp.exp(jnp.minimum(s + bias, 80.0))
        r = 1.0 / jnp.sum(e, axis=2, keepdims=True)
        agg = agg + lax.dot_general(e, wh, (((2,), (1,)), ((0,), (0,))),
                                    preferred_element_type=jnp.float32) * r

    h1 = jnp.einsum('bcd,hd->bch', agg, mw1a_ref[...],
                    preferred_element_type=jnp.float32)
    h1 = h1 + jnp.einsum('bcd,hd->bch', st, mw1s_ref[...],
                         preferred_element_type=jnp.float32)
    h1 = jnp.maximum(h1 + mb1_ref[...], 0.0)
    h1 = h1 * mgs1_ref[...][:, None, :] + mgb1_ref[...][:, None, :]
    msg = jnp.einsum('bch,dh->bcd', h1, mw2_ref[...],
                     preferred_element_type=jnp.float32) + mb2_ref[...]
    msg = msg * mgs2_ref[...][:, None, :] + mgb2_ref[...][:, None, :]

    h2 = jnp.einsum('bcd,hd->bch', st, sw1s_ref[...],
                    preferred_element_type=jnp.float32)
    h2 = h2 + jnp.einsum('bcd,hd->bch', msg, sw1m_ref[...],
                         preferred_element_type=jnp.float32)
    h2 = jnp.maximum(h2 + sb1_ref[...], 0.0)
    h2 = h2 * sgs1_ref[...][:, None, :] + sgb1_ref[...][:, None, :]
    delta = jnp.einsum('bch,dh->bcd', h2, sw2_ref[...],
                       preferred_element_type=jnp.float32) + sb2_ref[...]
    delta = delta * sgs2_ref[...][:, None, :] + sgb2_ref[...][:, None, :]

    out_ref[...] = st + delta


@jax.jit
def kernel(state, neuron_id, neuron_key,
           state_w1, state_b1, state_gs1, state_gb1, state_w2, state_b2,
           state_gs2, state_gb2,
           msg_w1, msg_b1, msg_gs1, msg_gb1, msg_w2, msg_b2, msg_gs2, msg_gb2,
           conn_idx, cell_to_group):
    g = cell_to_group
    mw1a = msg_w1[:, :D_N]
    mw1s = msg_w1[:, D_N:]
    sw1s = state_w1[:, :D_N]
    sw1m = state_w1[:, D_N:]
    mb1 = msg_b1.reshape(1, HM)
    mb2 = msg_b2.reshape(1, D_N)
    sb1 = state_b1.reshape(1, HS)
    sb2 = state_b2.reshape(1, D_N)
    mgs1, mgb1 = msg_gs1[g], msg_gb1[g]
    mgs2, mgb2 = msg_gs2[g], msg_gb2[g]
    sgs1, sgb1 = state_gs1[g], state_gb1[g]
    sgs2, sgb2 = state_gs2[g], state_gb2[g]

    grid = (N_CELLS // B,)
    cell_blk = pl.BlockSpec((B, C_N, D_N), lambda i: (i, 0, 0))
    conn_blk = pl.BlockSpec((B, C_N, K), lambda i: (i, 0, 0))

    def full2(a):
        return pl.BlockSpec(a.shape, lambda i: (0, 0))

    def row_blk(a):
        return pl.BlockSpec((B, a.shape[1]), lambda i: (i, 0))

    in_specs = [cell_blk, cell_blk, cell_blk, conn_blk,
                full2(mw1a), full2(mw1s), full2(mb1), row_blk(mgs1), row_blk(mgb1),
                full2(msg_w2), full2(mb2), row_blk(mgs2), row_blk(mgb2),
                full2(sw1s), full2(sw1m), full2(sb1), row_blk(sgs1), row_blk(sgb1),
                full2(state_w2), full2(sb2), row_blk(sgs2), row_blk(sgb2)]

    return pl.pallas_call(
        _fwd_kernel,
        grid=grid,
        in_specs=in_specs,
        out_specs=cell_blk,
        out_shape=jax.ShapeDtypeStruct((N_CELLS, C_N, D_N), jnp.float32),
    )(state, neuron_id, neuron_key, conn_idx,
      mw1a, mw1s, mb1, mgs1, mgb1, msg_w2, mb2, mgs2, mgb2,
      sw1s, sw1m, sb1, sgs1, sgb1, state_w2, sb2, sgs2, sgb2)


# speedup vs baseline: 46.4997x; 1.1030x over previous
import functools

import jax
import jax.numpy as jnp
from jax import lax
from jax.experimental import pallas as pl

D_N = 64
N_CELLS = 1024
C_N = 64
K = 16
HS = 128
HM = 128
HEADS = 4
DH = D_N // HEADS

B = 32


def _fwd_kernel(state_ref, nid_ref, nk_ref, conn_ref,
                mw1a_ref, mw1s_ref, mb1_ref, mgs1_ref, mgb1_ref,
                mw2_ref, mb2_ref, mgs2_ref, mgb2_ref,
                sw1s_ref, sw1m_ref, sb1_ref, sgs1_ref, sgb1_ref,
                sw2_ref, sb2_ref, sgs2_ref, sgb2_ref,
                out_ref):
    st = state_ref[...]
    x = st + nid_ref[...]
    q = nk_ref[...]
    idx = conn_ref[...]

    jf = lax.broadcasted_iota(jnp.int32, (B, C_N, C_N), 2).astype(jnp.float32)
    idxf = idx.astype(jnp.float32)
    p = idxf[:, :, 0][:, :, None] - jf
    for k in range(1, K):
        p = p * (idxf[:, :, k][:, :, None] - jf)
    nb = jnp.where(p == 0.0, 1.0, 0.0)

    h_iota = lax.broadcasted_iota(jnp.int32, (HEADS, D_N), 0)
    d_head = lax.broadcasted_iota(jnp.int32, (HEADS, D_N), 1) // DH
    hm = h_iota == d_head

    q = q * (1.0 / (DH ** 0.5))
    agg = jnp.zeros_like(st)
    for h in range(HEADS):
        wh = jnp.where(hm[h][None, None, :], x, 0.0)
        s = lax.dot_general(q, wh, (((2,), (2,)), ((0,), (0,))),
                            preferred_element_type=jnp.float32)
        e = jnp.exp(jnp.minimum(s, 80.0)) * nb
        r = 1.0 / jnp.sum(e, axis=2, keepdims=True)
        agg = agg + lax.dot_general(e, wh, (((2,), (1,)), ((0,), (0,))),
                                    preferred_element_type=jnp.float32) * r

    h1 = jnp.einsum('bcd,hd->bch', agg, mw1a_ref[...],
                    preferred_element_type=jnp.float32)
    h1 = h1 + jnp.einsum('bcd,hd->bch', st, mw1s_ref[...],
                         preferred_element_type=jnp.float32)
    h1 = jnp.maximum(h1 + mb1_ref[...], 0.0)
    h1 = h1 * mgs1_ref[...][:, None, :] + mgb1_ref[...][:, None, :]
    msg = jnp.einsum('bch,dh->bcd', h1, mw2_ref[...],
                     preferred_element_type=jnp.float32) + mb2_ref[...]
    msg = msg * mgs2_ref[...][:, None, :] + mgb2_ref[...][:, None, :]

    h2 = jnp.einsum('bcd,hd->bch', st, sw1s_ref[...],
                    preferred_element_type=jnp.float32)
    h2 = h2 + jnp.einsum('bcd,hd->bch', msg, sw1m_ref[...],
                         preferred_element_type=jnp.float32)
    h2 = jnp.maximum(h2 + sb1_ref[...], 0.0)
    h2 = h2 * sgs1_ref[...][:, None, :] + sgb1_ref[...][:, None, :]
    delta = jnp.einsum('bch,dh->bcd', h2, sw2_ref[...],
                       preferred_element_type=jnp.float32) + sb2_ref[...]
    delta = delta * sgs2_ref[...][:, None, :] + sgb2_ref[...][:, None, :]

    out_ref[...] = st + delta


@jax.jit
def kernel(state, neuron_id, neuron_key,
           state_w1, state_b1, state_gs1, state_gb1, state_w2, state_b2,
           state_gs2, state_gb2,
           msg_w1, msg_b1, msg_gs1, msg_gb1, msg_w2, msg_b2, msg_gs2, msg_gb2,
           conn_idx, cell_to_group):
    g = cell_to_group
    mw1a = msg_w1[:, :D_N]
    mw1s = msg_w1[:, D_N:]
    sw1s = state_w1[:, :D_N]
    sw1m = state_w1[:, D_N:]
    mb1 = msg_b1.reshape(1, HM)
    mb2 = msg_b2.reshape(1, D_N)
    sb1 = state_b1.reshape(1, HS)
    sb2 = state_b2.reshape(1, D_N)
    mgs1, mgb1 = msg_gs1[g], msg_gb1[g]
    mgs2, mgb2 = msg_gs2[g], msg_gb2[g]
    sgs1, sgb1 = state_gs1[g], state_gb1[g]
    sgs2, sgb2 = state_gs2[g], state_gb2[g]

    grid = (N_CELLS // B,)
    cell_blk = pl.BlockSpec((B, C_N, D_N), lambda i: (i, 0, 0))
    conn_blk = pl.BlockSpec((B, C_N, K), lambda i: (i, 0, 0))

    def full2(a):
        return pl.BlockSpec(a.shape, lambda i: (0, 0))

    def row_blk(a):
        return pl.BlockSpec((B, a.shape[1]), lambda i: (i, 0))

    in_specs = [cell_blk, cell_blk, cell_blk, conn_blk,
                full2(mw1a), full2(mw1s), full2(mb1), row_blk(mgs1), row_blk(mgb1),
                full2(msg_w2), full2(mb2), row_blk(mgs2), row_blk(mgb2),
                full2(sw1s), full2(sw1m), full2(sb1), row_blk(sgs1), row_blk(sgb1),
                full2(state_w2), full2(sb2), row_blk(sgs2), row_blk(sgb2)]

    return pl.pallas_call(
        _fwd_kernel,
        grid=grid,
        in_specs=in_specs,
        out_specs=cell_blk,
        out_shape=jax.ShapeDtypeStruct((N_CELLS, C_N, D_N), jnp.float32),
    )(state, neuron_id, neuron_key, conn_idx,
      mw1a, mw1s, mb1, mgs1, mgb1, msg_w2, mb2, mgs2, mgb2,
      sw1s, sw1m, sb1, sgs1, sgb1, state_w2, sb2, sgs2, sgb2)


# speedup vs baseline: 54.0856x; 1.1631x over previous
import functools

import jax
import jax.numpy as jnp
from jax import lax
from jax.experimental import pallas as pl
from jax.experimental.pallas import tpu as pltpu
from jax.experimental.pallas import tpu_sc as plsc

D_N = 64
N_CELLS = 1024
C_N = 64
K = 16
HS = 128
HM = 128
HEADS = 4
DH = D_N // HEADS

B = 32
N_ROWS = N_CELLS * C_N

NC = 2
NS = 16
NW = NC * NS
ROWS_PER_W = N_ROWS // NW
CHUNK = 512
NCHUNK = ROWS_PER_W // CHUNK


def _mask_sc_kernel(conn_hbm, out_hbm, idx_v, buf_v, ones_v, zeros_v):
    wid = lax.axis_index("s") * NC + lax.axis_index("c")

    ones = jnp.ones((16,), jnp.float32)
    zero16 = jnp.zeros((16,), jnp.float32)
    ones_v[...] = ones
    zeros_v[...] = zero16

    def _zero(i, _):
        buf_v[pl.ds(i * 16, 16)] = zeros_v[...]
        return 0

    lax.fori_loop(0, CHUNK * D_N // 16, _zero, 0)

    def _scatter(i, val):
        flat = idx_v[pl.ds(i * K, K)] + i * D_N
        plsc.store_scatter(buf_v, [flat], val)
        return flat

    for chunk in range(NCHUNK):
        base = wid * ROWS_PER_W + chunk * CHUNK
        pltpu.sync_copy(conn_hbm.at[pl.ds(base * K, CHUNK * K)], idx_v)

        def _set(i, _):
            _scatter(i, ones_v[...])
            return 0

        lax.fori_loop(0, CHUNK, _set, 0)
        pltpu.sync_copy(buf_v, out_hbm.at[pl.ds(base * D_N, CHUNK * D_N)])

        def _clr(i, _):
            _scatter(i, zeros_v[...])
            return 0

        lax.fori_loop(0, CHUNK, _clr, 0)


def _build_mask(conn_idx):
    k = functools.partial(
        pl.kernel,
        mesh=plsc.VectorSubcoreMesh(core_axis_name="c", subcore_axis_name="s"),
        out_type=jax.ShapeDtypeStruct((N_ROWS * D_N,), jnp.float32),
        scratch_types=[
            pltpu.VMEM((CHUNK * K,), jnp.int32),
            pltpu.VMEM((CHUNK * D_N,), jnp.float32),
            pltpu.VMEM((16,), jnp.float32),
            pltpu.VMEM((16,), jnp.float32),
        ],
        compiler_params=pltpu.CompilerParams(needs_layout_passes=False),
    )(_mask_sc_kernel)
    nb = k(conn_idx.reshape(-1))
    return nb.reshape(N_CELLS, C_N, C_N)


def _fwd_kernel(state_ref, nid_ref, nk_ref, nb_ref,
                mw1a_ref, mw1s_ref, mb1_ref, mgs1_ref, mgb1_ref,
                mw2_ref, mb2_ref, mgs2_ref, mgb2_ref,
                sw1s_ref, sw1m_ref, sb1_ref, sgs1_ref, sgb1_ref,
                sw2_ref, sb2_ref, sgs2_ref, sgb2_ref,
                out_ref):
    st = state_ref[...]
    x = st + nid_ref[...]
    q = nk_ref[...]
    nb = nb_ref[...]

    h_iota = lax.broadcasted_iota(jnp.int32, (HEADS, D_N), 0)
    d_head = lax.broadcasted_iota(jnp.int32, (HEADS, D_N), 1) // DH
    hm = h_iota == d_head

    q = q * (1.0 / (DH ** 0.5))
    agg = jnp.zeros_like(st)
    for h in range(HEADS):
        wh = jnp.where(hm[h][None, None, :], x, 0.0)
        s = lax.dot_general(q, wh, (((2,), (2,)), ((0,), (0,))),
                            preferred_element_type=jnp.float32)
        e = jnp.exp(jnp.minimum(s, 80.0)) * nb
        r = 1.0 / jnp.sum(e, axis=2, keepdims=True)
        agg = agg + lax.dot_general(e, wh, (((2,), (1,)), ((0,), (0,))),
                                    preferred_element_type=jnp.float32) * r

    h1 = jnp.einsum('bcd,hd->bch', agg, mw1a_ref[...],
                    preferred_element_type=jnp.float32)
    h1 = h1 + jnp.einsum('bcd,hd->bch', st, mw1s_ref[...],
                         preferred_element_type=jnp.float32)
    h1 = jnp.maximum(h1 + mb1_ref[...], 0.0)
    h1 = h1 * mgs1_ref[...][:, None, :] + mgb1_ref[...][:, None, :]
    msg = jnp.einsum('bch,dh->bcd', h1, mw2_ref[...],
                     preferred_element_type=jnp.float32) + mb2_ref[...]
    msg = msg * mgs2_ref[...][:, None, :] + mgb2_ref[...][:, None, :]

    h2 = jnp.einsum('bcd,hd->bch', st, sw1s_ref[...],
                    preferred_element_type=jnp.float32)
    h2 = h2 + jnp.einsum('bcd,hd->bch', msg, sw1m_ref[...],
                         preferred_element_type=jnp.float32)
    h2 = jnp.maximum(h2 + sb1_ref[...], 0.0)
    h2 = h2 * sgs1_ref[...][:, None, :] + sgb1_ref[...][:, None, :]
    delta = jnp.einsum('bch,dh->bcd', h2, sw2_ref[...],
                       preferred_element_type=jnp.float32) + sb2_ref[...]
    delta = delta * sgs2_ref[...][:, None, :] + sgb2_ref[...][:, None, :]

    out_ref[...] = st + delta


@jax.jit
def kernel(state, neuron_id, neuron_key,
           state_w1, state_b1, state_gs1, state_gb1, state_w2, state_b2,
           state_gs2, state_gb2,
           msg_w1, msg_b1, msg_gs1, msg_gb1, msg_w2, msg_b2, msg_gs2, msg_gb2,
           conn_idx, cell_to_group):
    g = cell_to_group
    mw1a = msg_w1[:, :D_N]
    mw1s = msg_w1[:, D_N:]
    sw1s = state_w1[:, :D_N]
    sw1m = state_w1[:, D_N:]
    mb1 = msg_b1.reshape(1, HM)
    mb2 = msg_b2.reshape(1, D_N)
    sb1 = state_b1.reshape(1, HS)
    sb2 = state_b2.reshape(1, D_N)
    mgs1, mgb1 = msg_gs1[g], msg_gb1[g]
    mgs2, mgb2 = msg_gs2[g], msg_gb2[g]
    sgs1, sgb1 = state_gs1[g], state_gb1[g]
    sgs2, sgb2 = state_gs2[g], state_gb2[g]

    nb = _build_mask(conn_idx)

    grid = (N_CELLS // B,)
    cell_blk = pl.BlockSpec((B, C_N, D_N), lambda i: (i, 0, 0))

    def full2(a):
        return pl.BlockSpec(a.shape, lambda i: (0, 0))

    def row_blk(a):
        return pl.BlockSpec((B, a.shape[1]), lambda i: (i, 0))

    in_specs = [cell_blk, cell_blk, cell_blk, cell_blk,
                full2(mw1a), full2(mw1s), full2(mb1), row_blk(mgs1), row_blk(mgb1),
                full2(msg_w2), full2(mb2), row_blk(mgs2), row_blk(mgb2),
                full2(sw1s), full2(sw1m), full2(sb1), row_blk(sgs1), row_blk(sgb1),
                full2(state_w2), full2(sb2), row_blk(sgs2), row_blk(sgb2)]

    return pl.pallas_call(
        _fwd_kernel,
        grid=grid,
        in_specs=in_specs,
        out_specs=cell_blk,
        out_shape=jax.ShapeDtypeStruct((N_CELLS, C_N, D_N), jnp.float32),
    )(state, neuron_id, neuron_key, nb,
      mw1a, mw1s, mb1, mgs1, mgb1, msg_w2, mb2, mgs2, mgb2,
      sw1s, sw1m, sb1, sgs1, sgb1, state_w2, sb2, sgs2, sgb2)


# speedup vs baseline: 54.1991x; 1.0021x over previous
import functools

import jax
import jax.numpy as jnp
from jax import lax
from jax.experimental import pallas as pl
from jax.experimental.pallas import tpu as pltpu
from jax.experimental.pallas import tpu_sc as plsc

D_N = 64
N_CELLS = 1024
C_N = 64
K = 16
HS = 128
HM = 128
HEADS = 4
DH = D_N // HEADS

B = 32
N_ROWS = N_CELLS * C_N

NC = 2
NS = 16
NW = NC * NS
ROWS_PER_W = N_ROWS // NW
CHUNK = 512
NCHUNK = ROWS_PER_W // CHUNK


def _mask_sc_kernel(conn_hbm, out_hbm, idx_v, buf_v):
    wid = lax.axis_index("s") * NC + lax.axis_index("c")

    ones = jnp.ones((16,), jnp.float32)
    zero16 = jnp.zeros((16,), jnp.float32)

    @plsc.parallel_loop(0, CHUNK * D_N // 16, unroll=8)
    def _zero(i):
        buf_v[pl.ds(i * 16, 16)] = zero16

    for chunk in range(NCHUNK):
        base = wid * ROWS_PER_W + chunk * CHUNK
        pltpu.sync_copy(conn_hbm.at[pl.ds(base * K, CHUNK * K)], idx_v)

        @plsc.parallel_loop(0, CHUNK, unroll=8)
        def _set(i):
            flat = idx_v[pl.ds(i * K, K)] + i * D_N
            plsc.store_scatter(buf_v, [flat], ones)

        pltpu.sync_copy(buf_v, out_hbm.at[pl.ds(base * D_N, CHUNK * D_N)])

        @plsc.parallel_loop(0, CHUNK, unroll=8)
        def _clr(i):
            flat = idx_v[pl.ds(i * K, K)] + i * D_N
            plsc.store_scatter(buf_v, [flat], zero16)


def _build_mask(conn_idx):
    k = functools.partial(
        pl.kernel,
        mesh=plsc.VectorSubcoreMesh(core_axis_name="c", subcore_axis_name="s"),
        out_type=jax.ShapeDtypeStruct((N_ROWS * D_N,), jnp.float32),
        scratch_types=[
            pltpu.VMEM((CHUNK * K,), jnp.int32),
            pltpu.VMEM((CHUNK * D_N,), jnp.float32),
        ],
        compiler_params=pltpu.CompilerParams(needs_layout_passes=False),
    )(_mask_sc_kernel)
    nb = k(conn_idx.reshape(-1))
    return nb.reshape(N_CELLS, C_N, C_N)


def _fwd_kernel(state_ref, nid_ref, nk_ref, nb_ref,
                mw1a_ref, mw1s_ref, mb1_ref, mgs1_ref, mgb1_ref,
                mw2_ref, mb2_ref, mgs2_ref, mgb2_ref,
                sw1s_ref, sw1m_ref, sb1_ref, sgs1_ref, sgb1_ref,
                sw2_ref, sb2_ref, sgs2_ref, sgb2_ref,
                out_ref):
    st = state_ref[...]
    x = st + nid_ref[...]
    q = nk_ref[...]
    nb = nb_ref[...]

    h_iota = lax.broadcasted_iota(jnp.int32, (HEADS, D_N), 0)
    d_head = lax.broadcasted_iota(jnp.int32, (HEADS, D_N), 1) // DH
    hm = h_iota == d_head

    q = q * (1.0 / (DH ** 0.5))
    agg = jnp.zeros_like(st)
    for h in range(HEADS):
        wh = jnp.where(hm[h][None, None, :], x, 0.0)
        s = lax.dot_general(q, wh, (((2,), (2,)), ((0,), (0,))),
                            preferred_element_type=jnp.float32)
        e = jnp.exp(jnp.minimum(s, 80.0)) * nb
        r = 1.0 / jnp.sum(e, axis=2, keepdims=True)
        agg = agg + lax.dot_general(e, wh, (((2,), (1,)), ((0,), (0,))),
                                    preferred_element_type=jnp.float32) * r

    h1 = jnp.einsum('bcd,hd->bch', agg, mw1a_ref[...],
                    preferred_element_type=jnp.float32)
    h1 = h1 + jnp.einsum('bcd,hd->bch', st, mw1s_ref[...],
                         preferred_element_type=jnp.float32)
    h1 = jnp.maximum(h1 + mb1_ref[...], 0.0)
    h1 = h1 * mgs1_ref[...][:, None, :] + mgb1_ref[...][:, None, :]
    msg = jnp.einsum('bch,dh->bcd', h1, mw2_ref[...],
                     preferred_element_type=jnp.float32) + mb2_ref[...]
    msg = msg * mgs2_ref[...][:, None, :] + mgb2_ref[...][:, None, :]

    h2 = jnp.einsum('bcd,hd->bch', st, sw1s_ref[...],
                    preferred_element_type=jnp.float32)
    h2 = h2 + jnp.einsum('bcd,hd->bch', msg, sw1m_ref[...],
                         preferred_element_type=jnp.float32)
    h2 = jnp.maximum(h2 + sb1_ref[...], 0.0)
    h2 = h2 * sgs1_ref[...][:, None, :] + sgb1_ref[...][:, None, :]
    delta = jnp.einsum('bch,dh->bcd', h2, sw2_ref[...],
                       preferred_element_type=jnp.float32) + sb2_ref[...]
    delta = delta * sgs2_ref[...][:, None, :] + sgb2_ref[...][:, None, :]

    out_ref[...] = st + delta


@jax.jit
def kernel(state, neuron_id, neuron_key,
           state_w1, state_b1, state_gs1, state_gb1, state_w2, state_b2,
           state_gs2, state_gb2,
           msg_w1, msg_b1, msg_gs1, msg_gb1, msg_w2, msg_b2, msg_gs2, msg_gb2,
           conn_idx, cell_to_group):
    g = cell_to_group
    mw1a = msg_w1[:, :D_N]
    mw1s = msg_w1[:, D_N:]
    sw1s = state_w1[:, :D_N]
    sw1m = state_w1[:, D_N:]
    mb1 = msg_b1.reshape(1, HM)
    mb2 = msg_b2.reshape(1, D_N)
    sb1 = state_b1.reshape(1, HS)
    sb2 = state_b2.reshape(1, D_N)
    mgs1, mgb1 = msg_gs1[g], msg_gb1[g]
    mgs2, mgb2 = msg_gs2[g], msg_gb2[g]
    sgs1, sgb1 = state_gs1[g], state_gb1[g]
    sgs2, sgb2 = state_gs2[g], state_gb2[g]

    nb = _build_mask(conn_idx)

    grid = (N_CELLS // B,)
    cell_blk = pl.BlockSpec((B, C_N, D_N), lambda i: (i, 0, 0))

    def full2(a):
        return pl.BlockSpec(a.shape, lambda i: (0, 0))

    def row_blk(a):
        return pl.BlockSpec((B, a.shape[1]), lambda i: (i, 0))

    in_specs = [cell_blk, cell_blk, cell_blk, cell_blk,
                full2(mw1a), full2(mw1s), full2(mb1), row_blk(mgs1), row_blk(mgb1),
                full2(msg_w2), full2(mb2), row_blk(mgs2), row_blk(mgb2),
                full2(sw1s), full2(sw1m), full2(sb1), row_blk(sgs1), row_blk(sgb1),
                full2(state_w2), full2(sb2), row_blk(sgs2), row_blk(sgb2)]

    return pl.pallas_call(
        _fwd_kernel,
        grid=grid,
        in_specs=in_specs,
        out_specs=cell_blk,
        out_shape=jax.ShapeDtypeStruct((N_CELLS, C_N, D_N), jnp.float32),
    )(state, neuron_id, neuron_key, nb,
      mw1a, mw1s, mb1, mgs1, mgb1, msg_w2, mb2, mgs2, mgb2,
      sw1s, sw1m, sb1, sgs1, sgb1, state_w2, sb2, sgs2, sgb2)
